# baseline (speedup 1.0000x reference)
import jax
import jax.numpy as jnp
from jax import lax
from jax.experimental import pallas as pl
from jax.experimental.pallas import tpu as pltpu
from jax.experimental.pallas import tpu_sc as plsc

B, C, N, K, G = 2, 288, 4096, 16, 9
D = C // G
NCH = C // 16
CW = 640
QW = 384
NS = 16
PTS = N // NS
CH = 8
NCHUNK = PTS // CH
GP = 16
CL = 37120

_DN = lax.GatherDimensionNumbers(
    offset_dims=(), collapsed_slice_dims=(0,), start_index_map=(0,))


def _shuf(v, idx):
    return lax.gather(v, idx[:, None], _DN, (1,),
                      mode=lax.GatherScatterMode.PROMISE_IN_BOUNDS)


def _sc_body(comb, idxo, feat_out, part_out,
             own, idxb, gbuf, feat_st, cent_loc, sem):
    c_id = lax.axis_index("c")
    s_id = lax.axis_index("s")
    b = c_id
    iota16 = lax.iota(jnp.int32, 16)
    fzero = jnp.zeros((16,), jnp.float32)

    @pl.loop(0, CL // 16)
    def _zero(r):
        cent_loc[pl.ds(r * 16, 16)] = fzero

    n_base = b * N + s_id * PTS

    bmasks = [((iota16 >> t) & 1) == 0 for t in range(4)]

    bn9 = b * N * 9

    def _compute(p, iv, rb):
        qch = [own[p, pl.ds(c * 16, 16)] for c in range(NCH)]

        def _attk(k, carry):
            outs = []
            for g in range(G):
                r0 = gbuf[rb + k, pl.ds((2 * g) * 16, 16)]
                r1 = gbuf[rb + k, pl.ds((2 * g + 1) * 16, 16)]
                s = qch[2 * g] * r0 + qch[2 * g + 1] * r1
                for sh in (1, 2, 4, 8):
                    s = s + _shuf(s, iota16 ^ sh)
                outs.append(jnp.where(iota16 == k, s, carry[g]))
            return tuple(outs)

        atts = lax.fori_loop(0, K, _attk, (fzero,) * G)

        watts = []
        for g in range(G):
            a = atts[g]
            m = a
            for sh in (1, 2, 4, 8):
                m = jnp.maximum(m, _shuf(m, iota16 ^ sh))
            e = jnp.exp(a - m)
            s = e
            for sh in (1, 2, 4, 8):
                s = s + _shuf(s, iota16 ^ sh)
            watts.append(e / s)

        faccs = [fzero] * NCH
        for k in range(K):
            kf = jnp.full((16,), k, jnp.int32)
            sp = [_shuf(watts[g], kf) for g in range(G)]
            row = fzero
            for g in range(G):
                row = jnp.where(iota16 == g, sp[g], row)
            j9 = iv[k] * 9 - bn9
            cent_loc[pl.ds(j9, 16)] = cent_loc[pl.ds(j9, 16)] + row
            for c in range(NCH):
                v = gbuf[rb + k, pl.ds(C + c * 16, 16)]
                faccs[c] = faccs[c] + sp[c // 2] * v
        for c in range(NCH):
            feat_st[pl.ds(p * C + c * 16, 16)] = faccs[c]

    @pl.loop(0, NCHUNK)
    def _chunk(ch):
        row0 = n_base + ch * CH
        pltpu.sync_copy(comb.at[pl.ds(row0, CH), pl.ds(0, QW)], own)
        pltpu.sync_copy(idxo.at[pl.ds(row0 * K, CH * K)], idxb)
        bN = b * N

        @pl.loop(0, CH)
        def _off(r):
            sl = pl.ds(r * 16, 16)
            idxb[sl] = idxb[sl] + bN

        @pl.loop(0, CH // 2)
        def _pair(pp):
            p0 = 2 * pp
            pltpu.async_copy(
                comb.at[idxb.at[pl.ds(p0 * K, 2 * K)]], gbuf, sem).wait()
            _compute(p0, idxb[pl.ds(p0 * K, K)], 0)
            _compute(p0 + 1, idxb[pl.ds(p0 * K + K, K)], K)

        pltpu.sync_copy(feat_st, feat_out.at[pl.ds(row0 * C, CH * C)])

    w_id = c_id * NS + s_id
    pltpu.sync_copy(cent_loc, part_out.at[pl.ds(w_id * CL, CL)])


def _red_body(part, cent_out, tmp, acc):
    c_id = lax.axis_index("c")
    s_id = lax.axis_index("s")
    span = CL // NS
    base = s_id * span
    pltpu.sync_copy(part.at[pl.ds(c_id * NS * CL + base, span)], acc)

    @pl.loop(1, NS)
    def _slab(t):
        pltpu.sync_copy(part.at[pl.ds((c_id * NS + t) * CL + base, span)], tmp)

        @pl.loop(0, span // 16)
        def _row(r):
            sl = pl.ds(r * 16, 16)
            acc[sl] = acc[sl] + tmp[sl]

    pltpu.sync_copy(acc, cent_out.at[pl.ds(c_id * CL + base, span)])


@jax.jit
def _run(comb, idxo):
    mesh = plsc.VectorSubcoreMesh(core_axis_name="c", subcore_axis_name="s")
    main = pl.kernel(
        _sc_body,
        out_type=(
            jax.ShapeDtypeStruct((B * N * C,), jnp.float32),
            jax.ShapeDtypeStruct((B * NS * CL,), jnp.float32),
        ),
        mesh=mesh,
        scratch_types=[
            pltpu.VMEM((CH, QW), jnp.float32),
            pltpu.VMEM((CH * K,), jnp.int32),
            pltpu.VMEM((2 * K, CW), jnp.float32),
            pltpu.VMEM((CH * C,), jnp.float32),
            pltpu.VMEM((CL,), jnp.float32),
            pltpu.SemaphoreType.DMA,
        ],
    )
    feat_flat, part = main(comb, idxo)
    reduce = pl.kernel(
        _red_body,
        out_type=jax.ShapeDtypeStruct((B * CL,), jnp.float32),
        mesh=plsc.VectorSubcoreMesh(core_axis_name="c", subcore_axis_name="s"),
        scratch_types=[
            pltpu.VMEM((CL // NS,), jnp.float32),
            pltpu.VMEM((CL // NS,), jnp.float32),
        ],
    )
    cent = reduce(part)
    return feat_flat, cent


def kernel(query_xyz, support_xyz, query_mask, support_mask,
           queryandkey, value, idx_knn):
    comb = jnp.concatenate(
        [queryandkey.transpose(0, 2, 1), value.transpose(0, 2, 1),
         jnp.zeros((B, N, CW - 2 * C), jnp.float32)],
        axis=-1).reshape(B * N, CW)
    idxo = idx_knn.reshape(B * N * K)
    feat_flat, cent = _run(comb, idxo)
    feat = feat_flat.reshape(B, N, C).transpose(0, 2, 1)
    cent = cent.reshape(B, CL)[:, :N * G].reshape(B, N, G)
    return feat, cent.transpose(0, 2, 1)


# speedup vs baseline: 1.4560x; 1.4560x over previous
import jax
import jax.numpy as jnp
from jax import lax
from jax.experimental import pallas as pl
from jax.experimental.pallas import tpu as pltpu
from jax.experimental.pallas import tpu_sc as plsc

B, C, N, K, G = 2, 288, 4096, 16, 9
D = C // G
NCH = C // 16
CW = 640
QW = 384
NS = 16
PTS = N // NS
CH = 8
NCHUNK = PTS // CH
GP = 16
CL = 37120

_DN = lax.GatherDimensionNumbers(
    offset_dims=(), collapsed_slice_dims=(0,), start_index_map=(0,))


def _shuf(v, idx):
    return lax.gather(v, idx[:, None], _DN, (1,),
                      mode=lax.GatherScatterMode.PROMISE_IN_BOUNDS)


def _sc_body(comb, idxo, feat_out, part_out,
             own, idxb, gbuf, feat_st, cent_loc, sem):
    c_id = lax.axis_index("c")
    s_id = lax.axis_index("s")
    b = c_id
    iota16 = lax.iota(jnp.int32, 16)
    fzero = jnp.zeros((16,), jnp.float32)

    @pl.loop(0, CL // 16)
    def _zero(r):
        cent_loc[pl.ds(r * 16, 16)] = fzero

    n_base = b * N + s_id * PTS

    bmasks = [((iota16 >> t) & 1) == 0 for t in range(4)]

    def _compute(p, iv, gbuf):
        qch = [own[p, pl.ds(c * 16, 16)] for c in range(NCH)]

        watts = []
        for g in range(G):
            vs = []
            for k in range(K):
                r0 = gbuf[k, pl.ds((2 * g) * 16, 16)]
                r1 = gbuf[k, pl.ds((2 * g + 1) * 16, 16)]
                vs.append(qch[2 * g] * r0 + qch[2 * g + 1] * r1)
            for t in range(4):
                m = bmasks[t]
                sh = 1 << t
                nxt = []
                for i in range(len(vs) // 2):
                    a, bb = vs[2 * i], vs[2 * i + 1]
                    keep = jnp.where(m, a, bb)
                    give = jnp.where(m, bb, a)
                    nxt.append(keep + _shuf(give, iota16 ^ sh))
                vs = nxt
            a = vs[0]
            mx = a
            for sh in (1, 2, 4, 8):
                mx = jnp.maximum(mx, _shuf(mx, iota16 ^ sh))
            e = jnp.exp(a - mx)
            s = e
            for sh in (1, 2, 4, 8):
                s = s + _shuf(s, iota16 ^ sh)
            watts.append(e / s)

        faccs = [fzero] * NCH
        for k in range(K):
            kf = jnp.full((16,), k, jnp.int32)
            sp = [_shuf(watts[g], kf) for g in range(G)]
            row = fzero
            for g in range(G):
                row = jnp.where(iota16 == g, sp[g], row)
            j9 = iv[k] * 9
            cent_loc[pl.ds(j9, 16)] = cent_loc[pl.ds(j9, 16)] + row
            for c in range(NCH):
                v = gbuf[k, pl.ds(C + c * 16, 16)]
                faccs[c] = faccs[c] + sp[c // 2] * v
        for c in range(NCH):
            feat_st[pl.ds(p * C + c * 16, 16)] = faccs[c]

    @pl.loop(0, NCHUNK)
    def _chunk(ch):
        row0 = n_base + ch * CH
        pltpu.sync_copy(comb.at[pl.ds(row0, CH), pl.ds(0, QW)], own)
        pltpu.sync_copy(idxo.at[pl.ds(row0, CH)], idxb)
        @pl.loop(0, CH)
        def _point(p):
            iv = idxb[p]
            pltpu.async_copy(comb.at[iv + b * N], gbuf, sem).wait()
            _compute(p, iv, gbuf)

        pltpu.sync_copy(feat_st, feat_out.at[pl.ds(row0 * C, CH * C)])

    w_id = c_id * NS + s_id
    pltpu.sync_copy(cent_loc, part_out.at[pl.ds(w_id * CL, CL)])


def _red_body(part, cent_out, tmp, acc):
    c_id = lax.axis_index("c")
    s_id = lax.axis_index("s")
    span = CL // NS
    base = s_id * span
    pltpu.sync_copy(part.at[pl.ds(c_id * NS * CL + base, span)], acc)

    @pl.loop(1, NS)
    def _slab(t):
        pltpu.sync_copy(part.at[pl.ds((c_id * NS + t) * CL + base, span)], tmp)

        @pl.loop(0, span // 16)
        def _row(r):
            sl = pl.ds(r * 16, 16)
            acc[sl] = acc[sl] + tmp[sl]

    pltpu.sync_copy(acc, cent_out.at[pl.ds(c_id * CL + base, span)])


@jax.jit
def _run(comb, idxo):
    mesh = plsc.VectorSubcoreMesh(core_axis_name="c", subcore_axis_name="s")
    main = pl.kernel(
        _sc_body,
        out_type=(
            jax.ShapeDtypeStruct((B * N * C,), jnp.float32),
            jax.ShapeDtypeStruct((B * NS * CL,), jnp.float32),
        ),
        mesh=mesh,
        scratch_types=[
            pltpu.VMEM((CH, QW), jnp.float32),
            pltpu.VMEM((CH, K), jnp.int32),
            pltpu.VMEM((K, CW), jnp.float32),
            pltpu.VMEM((CH * C,), jnp.float32),
            pltpu.VMEM((CL,), jnp.float32),
            pltpu.SemaphoreType.DMA,
        ],
    )
    feat_flat, part = main(comb, idxo)
    reduce = pl.kernel(
        _red_body,
        out_type=jax.ShapeDtypeStruct((B * CL,), jnp.float32),
        mesh=plsc.VectorSubcoreMesh(core_axis_name="c", subcore_axis_name="s"),
        scratch_types=[
            pltpu.VMEM((CL // NS,), jnp.float32),
            pltpu.VMEM((CL // NS,), jnp.float32),
        ],
    )
    cent = reduce(part)
    return feat_flat, cent


def kernel(query_xyz, support_xyz, query_mask, support_mask,
           queryandkey, value, idx_knn):
    comb = jnp.concatenate(
        [queryandkey.transpose(0, 2, 1), value.transpose(0, 2, 1),
         jnp.zeros((B, N, CW - 2 * C), jnp.float32)],
        axis=-1).reshape(B * N, CW)
    idxo = idx_knn.reshape(B * N, K)
    feat_flat, cent = _run(comb, idxo)
    feat = feat_flat.reshape(B, N, C).transpose(0, 2, 1)
    cent = cent.reshape(B, CL)[:, :N * G].reshape(B, N, G)
    return feat, cent.transpose(0, 2, 1)


# speedup vs baseline: 1.4668x; 1.0075x over previous
import jax
import jax.numpy as jnp
from jax import lax
from jax.experimental import pallas as pl
from jax.experimental.pallas import tpu as pltpu
from jax.experimental.pallas import tpu_sc as plsc

B, C, N, K, G = 2, 288, 4096, 16, 9
D = C // G
NCH = C // 16
CW = 640
QW = 384
NS = 16
PTS = N // NS
CH = 16
NCHUNK = PTS // CH
GP = 16
CL = 37120

_DN = lax.GatherDimensionNumbers(
    offset_dims=(), collapsed_slice_dims=(0,), start_index_map=(0,))


def _shuf(v, idx):
    return lax.gather(v, idx[:, None], _DN, (1,),
                      mode=lax.GatherScatterMode.PROMISE_IN_BOUNDS)


def _sc_body(comb, idxo, feat_out, part_out,
             own, idxb, gbuf, feat_st, cent_loc, sem):
    c_id = lax.axis_index("c")
    s_id = lax.axis_index("s")
    b = c_id
    iota16 = lax.iota(jnp.int32, 16)
    fzero = jnp.zeros((16,), jnp.float32)

    @pl.loop(0, CL // 16)
    def _zero(r):
        cent_loc[pl.ds(r * 16, 16)] = fzero

    n_base = b * N + s_id * PTS

    bmasks = [((iota16 >> t) & 1) == 0 for t in range(4)]

    def _compute(p, iv, gbuf):
        qch = [own[p, pl.ds(c * 16, 16)] for c in range(NCH)]

        watts = []
        for g in range(G):
            vs = []
            for k in range(K):
                r0 = gbuf[k, pl.ds((2 * g) * 16, 16)]
                r1 = gbuf[k, pl.ds((2 * g + 1) * 16, 16)]
                vs.append(qch[2 * g] * r0 + qch[2 * g + 1] * r1)
            for t in range(4):
                m = bmasks[t]
                sh = 1 << t
                nxt = []
                for i in range(len(vs) // 2):
                    a, bb = vs[2 * i], vs[2 * i + 1]
                    keep = jnp.where(m, a, bb)
                    give = jnp.where(m, bb, a)
                    nxt.append(keep + _shuf(give, iota16 ^ sh))
                vs = nxt
            a = vs[0]
            mx = a
            for sh in (1, 2, 4, 8):
                mx = jnp.maximum(mx, _shuf(mx, iota16 ^ sh))
            e = jnp.exp(a - mx)
            s = e
            for sh in (1, 2, 4, 8):
                s = s + _shuf(s, iota16 ^ sh)
            watts.append(e / s)

        faccs = [fzero] * NCH
        for k in range(K):
            kf = jnp.full((16,), k, jnp.int32)
            sp = [_shuf(watts[g], kf) for g in range(G)]
            row = fzero
            for g in range(G):
                row = jnp.where(iota16 == g, sp[g], row)
            j9 = iv[k] * 9
            cent_loc[pl.ds(j9, 16)] = cent_loc[pl.ds(j9, 16)] + row
            for c in range(NCH):
                v = gbuf[k, pl.ds(C + c * 16, 16)]
                faccs[c] = faccs[c] + sp[c // 2] * v
        for c in range(NCH):
            feat_st[pl.ds(p * C + c * 16, 16)] = faccs[c]

    @pl.loop(0, NCHUNK)
    def _chunk(ch):
        row0 = n_base + ch * CH
        pltpu.sync_copy(comb.at[pl.ds(row0, CH), pl.ds(0, QW)], own)
        pltpu.sync_copy(idxo.at[pl.ds(row0, CH)], idxb)
        @pl.loop(0, CH)
        def _point(p):
            iv = idxb[p]
            pltpu.async_copy(comb.at[iv + b * N], gbuf, sem).wait()
            _compute(p, iv, gbuf)

        pltpu.sync_copy(feat_st, feat_out.at[pl.ds(row0 * C, CH * C)])

    w_id = c_id * NS + s_id
    pltpu.sync_copy(cent_loc, part_out.at[pl.ds(w_id * CL, CL)])


def _red_body(part, cent_out, tmp, acc):
    c_id = lax.axis_index("c")
    s_id = lax.axis_index("s")
    span = CL // NS
    base = s_id * span
    pltpu.sync_copy(part.at[pl.ds(c_id * NS * CL + base, span)], acc)

    @pl.loop(1, NS)
    def _slab(t):
        pltpu.sync_copy(part.at[pl.ds((c_id * NS + t) * CL + base, span)], tmp)

        @pl.loop(0, span // 16)
        def _row(r):
            sl = pl.ds(r * 16, 16)
            acc[sl] = acc[sl] + tmp[sl]

    pltpu.sync_copy(acc, cent_out.at[pl.ds(c_id * CL + base, span)])


@jax.jit
def _run(comb, idxo):
    mesh = plsc.VectorSubcoreMesh(core_axis_name="c", subcore_axis_name="s")
    main = pl.kernel(
        _sc_body,
        out_type=(
            jax.ShapeDtypeStruct((B * N * C,), jnp.float32),
            jax.ShapeDtypeStruct((B * NS * CL,), jnp.float32),
        ),
        mesh=mesh,
        scratch_types=[
            pltpu.VMEM((CH, QW), jnp.float32),
            pltpu.VMEM((CH, K), jnp.int32),
            pltpu.VMEM((K, CW), jnp.float32),
            pltpu.VMEM((CH * C,), jnp.float32),
            pltpu.VMEM((CL,), jnp.float32),
            pltpu.SemaphoreType.DMA,
        ],
    )
    feat_flat, part = main(comb, idxo)
    reduce = pl.kernel(
        _red_body,
        out_type=jax.ShapeDtypeStruct((B * CL,), jnp.float32),
        mesh=plsc.VectorSubcoreMesh(core_axis_name="c", subcore_axis_name="s"),
        scratch_types=[
            pltpu.VMEM((CL // NS,), jnp.float32),
            pltpu.VMEM((CL // NS,), jnp.float32),
        ],
    )
    cent = reduce(part)
    return feat_flat, cent


def kernel(query_xyz, support_xyz, query_mask, support_mask,
           queryandkey, value, idx_knn):
    comb = jnp.concatenate(
        [queryandkey.transpose(0, 2, 1), value.transpose(0, 2, 1),
         jnp.zeros((B, N, CW - 2 * C), jnp.float32)],
        axis=-1).reshape(B * N, CW)
    idxo = idx_knn.reshape(B * N, K)
    feat_flat, cent = _run(comb, idxo)
    feat = feat_flat.reshape(B, N, C).transpose(0, 2, 1)
    cent = cent.reshape(B, CL)[:, :N * G].reshape(B, N, G)
    return feat, cent.transpose(0, 2, 1)
